# Initial kernel scaffold; baseline (speedup 1.0000x reference)
import jax
import jax.numpy as jnp
from jax.experimental import pallas as pl


def kernel(hit_positions, hit_normals, hit_primIDs, vbo, ibo, features, normal_features, gain_dB):
    raise NotImplementedError("write your pallas kernel here")


# speedup vs baseline: 12.6068x; 12.6068x over previous
import functools
import math

import jax
import jax.numpy as jnp
from jax import lax
from jax.experimental import pallas as pl
from jax.experimental.pallas import tpu as pltpu, tpu_sc as plsc

NC = 2
NS = 16
NW = NC * NS

CHUNK = 256
JSL = CHUNK // 128

ROW = 128
PAY = 16

_LN10_OVER_5 = math.log(10.0) / 5.0


@functools.lru_cache(maxsize=2)
def _build_sc_gather(n_hits: int):
    hits_per_w = n_hits // NW
    n_chunks = hits_per_w // CHUNK
    mesh = plsc.VectorSubcoreMesh(core_axis_name="c", subcore_axis_name="s")
    row_t = jax.ShapeDtypeStruct((n_hits, ROW), jnp.float32)

    @functools.partial(
        pl.kernel,
        out_type=[row_t, row_t, row_t],
        mesh=mesh,
        scratch_types=[
            pltpu.VMEM((JSL, 128), jnp.int32),
            pltpu.VMEM((JSL, 128), jnp.int32),
            pltpu.VMEM((JSL, 128), jnp.int32),
            pltpu.VMEM((JSL, 128), jnp.int32),
            pltpu.VMEM((CHUNK, ROW), jnp.float32),
            pltpu.VMEM((CHUNK, ROW), jnp.float32),
            pltpu.VMEM((CHUNK, ROW), jnp.float32),
            pltpu.SemaphoreType.DMA,
        ],
    )
    def sc_gather(prim_hbm, iboa_hbm, ibob_hbm, iboc_hbm, tab_hbm,
                  rowa_hbm, rowb_hbm, rowc_hbm,
                  prim_v, ia_v, ib_v, ic_v, ra_v, rb_v, rc_v, sem):
        wid = lax.axis_index("s") * NC + lax.axis_index("c")
        base0 = wid * hits_per_w

        def chunk_body(ci, carry):
            base = base0 + ci * CHUNK
            pltpu.sync_copy(prim_hbm.at[wid * n_chunks + ci], prim_v)

            descs = []
            for j in range(JSL):
                idx = prim_v.at[j]
                descs.append(pltpu.async_copy(iboa_hbm.at[idx], ia_v.at[j], sem))
                descs.append(pltpu.async_copy(ibob_hbm.at[idx], ib_v.at[j], sem))
                descs.append(pltpu.async_copy(iboc_hbm.at[idx], ic_v.at[j], sem))
            for d in descs:
                d.wait()

            descs = []
            for j in range(JSL):
                sl = pl.ds(j * 128, 128)
                descs.append(pltpu.async_copy(tab_hbm.at[ia_v.at[j]], ra_v.at[sl], sem))
                descs.append(pltpu.async_copy(tab_hbm.at[ib_v.at[j]], rb_v.at[sl], sem))
                descs.append(pltpu.async_copy(tab_hbm.at[ic_v.at[j]], rc_v.at[sl], sem))
            for d in descs:
                d.wait()

            out_sl = pl.ds(base, CHUNK)
            pltpu.sync_copy(ra_v, rowa_hbm.at[out_sl])
            pltpu.sync_copy(rb_v, rowb_hbm.at[out_sl])
            pltpu.sync_copy(rc_v, rowc_hbm.at[out_sl])
            return carry

        lax.fori_loop(0, n_chunks, chunk_body, 0)

    return sc_gather


def _tc_math(n_hits, block,
             ra_ref, rb_ref, rc_ref, pos_ref, nrm_ref, gain_ref, o_ref):
    ra = ra_ref[...]
    rb = rb_ref[...]
    rc = rc_ref[...]
    p3 = pos_ref[...]

    a3 = ra[:, 0:3]
    v0 = rb[:, 0:3] - a3
    v1 = rc[:, 0:3] - a3
    v2 = p3 - a3
    d00 = jnp.sum(v0 * v0, axis=1, keepdims=True)
    d01 = jnp.sum(v0 * v1, axis=1, keepdims=True)
    d11 = jnp.sum(v1 * v1, axis=1, keepdims=True)
    d20 = jnp.sum(v2 * v0, axis=1, keepdims=True)
    d21 = jnp.sum(v2 * v1, axis=1, keepdims=True)
    denom = d00 * d11 - d01 * d01 + 1e-8
    v = (d11 * d20 - d01 * d21) / denom
    w = (d00 * d21 - d01 * d20) / denom
    u = 1.0 - v - w
    u = jnp.clip(u, 0.0, 1.0)
    v = jnp.clip(v, 0.0, 1.0)
    w = jnp.clip(w, 0.0, 1.0)
    rs = 1.0 / (u + v + w + 1e-8)
    wu = u * rs
    wv = v * rs
    ww = w * rs

    nm3 = wu * ra[:, 11:14] + wv * rb[:, 11:14] + ww * rc[:, 11:14]
    nv = nrm_ref[...] + (nm3 * 2.0 - 1.0) * 0.25
    norm = jnp.sqrt(jnp.sum(nv * nv, axis=1, keepdims=True))
    cosine = jnp.clip(nv[:, 2:3] / (norm + 1e-8), 0.0, 1.0)

    g = gain_ref[0, 0]
    scale = jnp.exp(g * _LN10_OVER_5) * (1.0 / n_hits)

    feat = wu * ra[:, 3:11] + wv * rb[:, 3:11] + ww * rc[:, 3:11]
    o_ref[...] = feat * (cosine * scale)


@functools.lru_cache(maxsize=2)
def _build_tc_call(n_hits: int, block: int):
    grid = n_hits // block
    row_spec = pl.BlockSpec((block, ROW), lambda i: (i, 0))
    vec_spec = pl.BlockSpec((block, 3), lambda i: (i, 0))
    return pl.pallas_call(
        functools.partial(_tc_math, n_hits, block),
        grid=(grid,),
        in_specs=[row_spec, row_spec, row_spec, vec_spec, vec_spec,
                  pl.BlockSpec((1, 1), lambda i: (0, 0))],
        out_specs=pl.BlockSpec((block, 8), lambda i: (i, 0)),
        out_shape=jax.ShapeDtypeStruct((n_hits, 8), jnp.float32),
    )


def kernel(hit_positions, hit_normals, hit_primIDs, vbo, ibo, features,
           normal_features, gain_dB):
    n_hits = hit_positions.shape[0]
    n_feat = features.shape[1]
    pad = ROW - 3 - n_feat - 3
    table = jnp.concatenate(
        [vbo, features, normal_features,
         jnp.zeros((vbo.shape[0], pad), jnp.float32)], axis=1)
    iboa = ibo[:, 0].astype(jnp.int32)
    ibob = ibo[:, 1].astype(jnp.int32)
    iboc = ibo[:, 2].astype(jnp.int32)
    prim3d = hit_primIDs.reshape(n_hits // CHUNK, JSL, 128).astype(jnp.int32)

    rows_a, rows_b, rows_c = _build_sc_gather(n_hits)(
        prim3d, iboa, ibob, iboc, table)

    g11 = gain_dB.astype(jnp.float32).reshape(1, 1)
    return _build_tc_call(n_hits, 2048)(
        rows_a, rows_b, rows_c, hit_positions, hit_normals, g11)
